# baseline (speedup 1.0000x reference)
import functools

import jax
import jax.numpy as jnp
from jax import lax
from jax.experimental import pallas as pl
from jax.experimental.pallas import tpu as pltpu
from jax.experimental.pallas import tpu_sc as plsc

VOCAB = 1000000
D = 64
L = 16
NC, NS = 2, 16
NW = NC * NS
CHUNK = 128


def _pe_row():
    i = jnp.arange(D // 2, dtype=jnp.float32)
    ij = i / jnp.power(10000.0, 2.0 * (i / D))
    sin_cos = jnp.stack([jnp.sin(ij), jnp.cos(ij)], axis=1)
    return jnp.reshape(sin_cos, (D,))


def _body(n_chunks_w, idx_hbm, pe_hbm, table_hbm, out_hbm,
          idx_v, pe_v, buf_v, sem):
    wid = lax.axis_index("s") * NC + lax.axis_index("c")
    base = wid * n_chunks_w
    pltpu.sync_copy(idx_hbm.at[wid], idx_v)
    pltpu.sync_copy(pe_hbm, pe_v)
    pe_regs = [pe_v[pl.ds(L * t, L)] for t in range(D // L)]

    @pl.loop(0, n_chunks_w)
    def _chunk(j):
        pltpu.async_copy(table_hbm.at[idx_v.at[j]], buf_v, sem).wait()

        @pl.loop(0, CHUNK)
        def _row(r):
            for t in range(D // L):
                plsc.addupdate(buf_v.at[r, pl.ds(L * t, L)], pe_regs[t])

        pltpu.sync_copy(buf_v, out_hbm.at[pl.ds((base + j) * CHUNK, CHUNK)])


def kernel(inputs, table):
    bsz, seq = inputs.shape
    n = bsz * seq
    assert n % (NW * CHUNK) == 0
    n_chunks_w = n // (NW * CHUNK)
    idx = inputs.reshape(-1).astype(jnp.int32).reshape(NW, n_chunks_w, CHUNK)
    pe = _pe_row()

    mesh = plsc.VectorSubcoreMesh(core_axis_name="c", subcore_axis_name="s")
    gather = pl.kernel(
        functools.partial(_body, n_chunks_w),
        out_type=jax.ShapeDtypeStruct((n, D), jnp.float32),
        mesh=mesh,
        compiler_params=pltpu.CompilerParams(use_tc_tiling_on_sc=False),
        scratch_types=[
            pltpu.VMEM((n_chunks_w, CHUNK), jnp.int32),
            pltpu.VMEM((D,), jnp.float32),
            pltpu.VMEM((CHUNK, D), jnp.float32),
            pltpu.SemaphoreType.DMA,
        ],
    )
    out = gather(idx, pe, table)
    return out.reshape(bsz, seq, D)


# speedup vs baseline: 1.0801x; 1.0801x over previous
import functools

import jax
import jax.numpy as jnp
from jax import lax
from jax.experimental import pallas as pl
from jax.experimental.pallas import tpu as pltpu
from jax.experimental.pallas import tpu_sc as plsc

VOCAB = 1000000
D = 64
L = 16
NC, NS = 2, 16
NW = NC * NS
CHUNK = 128


def _pe_row():
    i = jnp.arange(D // 2, dtype=jnp.float32)
    ij = i / jnp.power(10000.0, 2.0 * (i / D))
    sin_cos = jnp.stack([jnp.sin(ij), jnp.cos(ij)], axis=1)
    return jnp.reshape(sin_cos, (D,))


NBUF = 10
AHEAD = 5


def _body(n_chunks_w, idx_hbm, pe_hbm, table_hbm, out_hbm,
          idx_v, pe_v, bufs, gsem, wsem):
    wid = lax.axis_index("s") * NC + lax.axis_index("c")
    base = wid * n_chunks_w
    pltpu.sync_copy(idx_hbm.at[wid], idx_v)
    pltpu.sync_copy(pe_hbm, pe_v)
    pe_regs = [pe_v[pl.ds(L * t, L)] for t in range(D // L)]

    def fire_gather(chunk, b):
        pltpu.async_copy(table_hbm.at[idx_v.at[chunk]], bufs.at[b],
                         gsem.at[b])

    def wait_gather(chunk, b):
        pltpu.make_async_copy(table_hbm.at[idx_v.at[chunk]], bufs.at[b],
                              gsem.at[b]).wait()

    def fire_write(chunk, b):
        pltpu.async_copy(bufs.at[b], out_hbm.at[pl.ds((base + chunk) * CHUNK,
                                                      CHUNK)], wsem.at[b])

    def wait_write(b):
        pltpu.make_async_copy(bufs.at[b], out_hbm.at[pl.ds(0, CHUNK)],
                              wsem.at[b]).wait()

    for j in range(AHEAD):
        fire_gather(j, j % NBUF)

    @pl.loop(0, n_chunks_w, step=NBUF)
    def _group(j0):
        for b in range(NBUF):
            j = j0 + b
            k = j + AHEAD
            kb = (b + AHEAD) % NBUF

            @pl.when(k < n_chunks_w)
            def _():
                @pl.when(k >= NBUF)
                def _():
                    wait_write(kb)
                fire_gather(k, kb)

            wait_gather(j, b)

            @pl.loop(0, CHUNK, unroll=4)
            def _row(r):
                for t in range(D // L):
                    plsc.addupdate(bufs.at[b, r, pl.ds(L * t, L)], pe_regs[t])

            fire_write(j, b)

    for b in range(NBUF):
        wait_write(b)


def kernel(inputs, table):
    bsz, seq = inputs.shape
    n = bsz * seq
    assert n % (NW * CHUNK) == 0
    n_chunks_w = n // (NW * CHUNK)
    idx = inputs.reshape(-1).astype(jnp.int32).reshape(NW, n_chunks_w, CHUNK)
    pe = _pe_row()

    mesh = plsc.VectorSubcoreMesh(core_axis_name="c", subcore_axis_name="s")
    gather = pl.kernel(
        functools.partial(_body, n_chunks_w),
        out_type=jax.ShapeDtypeStruct((n, D), jnp.float32),
        mesh=mesh,
        compiler_params=pltpu.CompilerParams(use_tc_tiling_on_sc=False),
        scratch_types=[
            pltpu.VMEM((n_chunks_w, CHUNK), jnp.int32),
            pltpu.VMEM((D,), jnp.float32),
            pltpu.VMEM((NBUF, CHUNK, D), jnp.float32),
            pltpu.SemaphoreType.DMA((NBUF,)),
            pltpu.SemaphoreType.DMA((NBUF,)),
        ],
    )
    out = gather(idx, pe, table)
    return out.reshape(bsz, seq, D)
